# baseline (speedup 1.0000x reference)
import functools

import jax
import jax.numpy as jnp
from jax import lax
from jax.experimental import pallas as pl
from jax.experimental.pallas import tpu as pltpu
from jax.experimental.pallas import tpu_sc as plsc

L = 16


@functools.lru_cache(maxsize=None)
def _build_sc_gather(sc_rows, d, f, chunk_rows, row_base):
    info = plsc.get_sparse_core_info()
    nc, ns = info.num_cores, info.num_subcores
    nw = nc * ns
    assert sc_rows % (nw * chunk_rows) == 0
    rows_per_worker = sc_rows // nw
    n_chunks = rows_per_worker // chunk_rows
    n_groups = f // L

    mesh = plsc.VectorSubcoreMesh(core_axis_name="c", subcore_axis_name="s")

    assert n_chunks % 2 == 0
    n_pairs = n_chunks // 2

    @functools.partial(
        pl.kernel,
        out_type=jax.ShapeDtypeStruct((sc_rows, f), jnp.float32),
        mesh=mesh,
        compiler_params=pltpu.CompilerParams(needs_layout_passes=False),
        scratch_types=[
            pltpu.VMEM((f,), jnp.int32),
            pltpu.VMEM((chunk_rows, d), jnp.float32),
            pltpu.VMEM((chunk_rows, d), jnp.float32),
            pltpu.VMEM((chunk_rows, f), jnp.float32),
            pltpu.VMEM((chunk_rows, f), jnp.float32),
            pltpu.SemaphoreType.DMA,
            pltpu.SemaphoreType.DMA,
            pltpu.SemaphoreType.DMA,
            pltpu.SemaphoreType.DMA,
        ],
    )
    def sc_gather(x_hbm, idx_hbm, out_hbm, idx_v, in0, in1, ob0, ob1,
                  si0, si1, so0, so1):
        wid = lax.axis_index("s") * nc + lax.axis_index("c")
        base = wid * rows_per_worker
        pltpu.sync_copy(idx_hbm, idx_v)
        col_idx = [idx_v[pl.ds(g * L, L)] for g in range(n_groups)]

        def cp_in(c, buf, sem):
            return pltpu.make_async_copy(
                x_hbm.at[pl.ds(row_base + base + c * chunk_rows, chunk_rows)],
                buf, sem)

        def cp_out(c, buf, sem):
            return pltpu.make_async_copy(
                buf, out_hbm.at[pl.ds(base + c * chunk_rows, chunk_rows)], sem)

        def subsample(in_b, out_b):
            for i in range(chunk_rows):
                row = jnp.full((L,), i, jnp.int32)
                for g in range(n_groups):
                    vals = plsc.load_gather(in_b, [row, col_idx[g]])
                    out_b[i, pl.ds(g * L, L)] = vals

        cp_in(0, in0, si0).start()
        cp_in(1, in1, si1).start()

        def body(k, carry):
            c0 = 2 * k
            cp_in(c0, in0, si0).wait()

            @pl.when(k > 0)
            def _():
                cp_out(c0 - 2, ob0, so0).wait()

            subsample(in0, ob0)
            cp_out(c0, ob0, so0).start()

            @pl.when(k < n_pairs - 1)
            def _():
                cp_in(c0 + 2, in0, si0).start()

            cp_in(c0 + 1, in1, si1).wait()

            @pl.when(k > 0)
            def _():
                cp_out(c0 - 1, ob1, so1).wait()

            subsample(in1, ob1)
            cp_out(c0 + 1, ob1, so1).start()

            @pl.when(k < n_pairs - 1)
            def _():
                cp_in(c0 + 3, in1, si1).start()

            return carry

        lax.fori_loop(0, n_pairs, body, 0)
        cp_out(n_chunks - 2, ob0, so0).wait()
        cp_out(n_chunks - 1, ob1, so1).wait()

    return sc_gather


@functools.lru_cache(maxsize=None)
def _build_tc_gather(nrows_tc, d, f, br, nrows_out):
    def body(idx_ref, x_ref, o_ref, oh_ref):
        @pl.when(pl.program_id(0) == 0)
        def _():
            di = lax.broadcasted_iota(jnp.int32, (d, f), 0)
            oh_ref[...] = jnp.where(di == idx_ref[0, :][None, :], 1.0, 0.0)

        o_ref[...] = jnp.dot(x_ref[...], oh_ref[...],
                             preferred_element_type=jnp.float32)

    return pl.pallas_call(
        body,
        grid=(nrows_tc // br,),
        in_specs=[
            pl.BlockSpec((8, f), lambda i: (0, 0)),
            pl.BlockSpec((br, d), lambda i: (i, 0)),
        ],
        out_specs=pl.BlockSpec((br, f), lambda i: (i, 0)),
        out_shape=jax.ShapeDtypeStruct((nrows_out, f), jnp.float32),
        scratch_shapes=[pltpu.VMEM((d, f), jnp.float32)],
    )


_SC_ROWS = 4096


def kernel(X, feature_indices):
    b, s, d = X.shape
    f = feature_indices.shape[0]
    nrows = b * s
    x2d = X.reshape(nrows, d)
    tc_rows = nrows - _SC_ROWS
    idx8 = jnp.broadcast_to(feature_indices, (8, f))
    out_sc = _build_sc_gather(_SC_ROWS, d, f, 16, tc_rows)(x2d, feature_indices)
    out_full = _build_tc_gather(tc_rows, d, f, 512, nrows)(idx8, x2d)
    out2d = lax.dynamic_update_slice(out_full, out_sc, (tc_rows, 0))
    return out2d.reshape(b, s, f)


# speedup vs baseline: 1.0363x; 1.0363x over previous
import functools

import jax
import jax.numpy as jnp
from jax import lax
from jax.experimental import pallas as pl
from jax.experimental.pallas import tpu as pltpu
from jax.experimental.pallas import tpu_sc as plsc

L = 16


@functools.lru_cache(maxsize=None)
def _build_sc_gather(sc_rows, d, f, chunk_rows, row_base):
    info = plsc.get_sparse_core_info()
    nc, ns = info.num_cores, info.num_subcores
    nw = nc * ns
    assert sc_rows % (nw * chunk_rows) == 0
    rows_per_worker = sc_rows // nw
    n_chunks = rows_per_worker // chunk_rows
    n_groups = f // L

    mesh = plsc.VectorSubcoreMesh(core_axis_name="c", subcore_axis_name="s")

    assert n_chunks % 2 == 0
    n_pairs = n_chunks // 2

    @functools.partial(
        pl.kernel,
        out_type=jax.ShapeDtypeStruct((sc_rows, f), jnp.float32),
        mesh=mesh,
        compiler_params=pltpu.CompilerParams(needs_layout_passes=False),
        scratch_types=[
            pltpu.VMEM((f,), jnp.int32),
            pltpu.VMEM((chunk_rows, d), jnp.float32),
            pltpu.VMEM((chunk_rows, d), jnp.float32),
            pltpu.VMEM((chunk_rows, f), jnp.float32),
            pltpu.VMEM((chunk_rows, f), jnp.float32),
            pltpu.SemaphoreType.DMA,
            pltpu.SemaphoreType.DMA,
            pltpu.SemaphoreType.DMA,
            pltpu.SemaphoreType.DMA,
        ],
    )
    def sc_gather(x_hbm, idx_hbm, out_hbm, idx_v, in0, in1, ob0, ob1,
                  si0, si1, so0, so1):
        wid = lax.axis_index("s") * nc + lax.axis_index("c")
        base = wid * rows_per_worker
        pltpu.sync_copy(idx_hbm, idx_v)
        col_idx = [idx_v[pl.ds(g * L, L)] for g in range(n_groups)]

        def cp_in(c, buf, sem):
            return pltpu.make_async_copy(
                x_hbm.at[pl.ds(row_base + base + c * chunk_rows, chunk_rows)],
                buf, sem)

        def cp_out(c, buf, sem):
            return pltpu.make_async_copy(
                buf, out_hbm.at[pl.ds(base + c * chunk_rows, chunk_rows)], sem)

        def subsample(in_b, out_b):
            def row_body(i, carry):
                row = jnp.full((L,), i, jnp.int32)
                for g in range(n_groups):
                    vals = plsc.load_gather(in_b, [row, col_idx[g]])
                    out_b[i, pl.ds(g * L, L)] = vals
                return carry

            lax.fori_loop(0, chunk_rows, row_body, 0)

        cp_in(0, in0, si0).start()
        cp_in(1, in1, si1).start()

        def body(k, carry):
            c0 = 2 * k
            cp_in(c0, in0, si0).wait()

            @pl.when(k > 0)
            def _():
                cp_out(c0 - 2, ob0, so0).wait()

            subsample(in0, ob0)
            cp_out(c0, ob0, so0).start()

            @pl.when(k < n_pairs - 1)
            def _():
                cp_in(c0 + 2, in0, si0).start()

            cp_in(c0 + 1, in1, si1).wait()

            @pl.when(k > 0)
            def _():
                cp_out(c0 - 1, ob1, so1).wait()

            subsample(in1, ob1)
            cp_out(c0 + 1, ob1, so1).start()

            @pl.when(k < n_pairs - 1)
            def _():
                cp_in(c0 + 3, in1, si1).start()

            return carry

        lax.fori_loop(0, n_pairs, body, 0)
        cp_out(n_chunks - 2, ob0, so0).wait()
        cp_out(n_chunks - 1, ob1, so1).wait()

    return sc_gather


@functools.lru_cache(maxsize=None)
def _build_tc_gather(nrows_tc, d, f, br, nrows_out):
    def body(idx_ref, x_ref, o_ref, oh_ref):
        @pl.when(pl.program_id(0) == 0)
        def _():
            di = lax.broadcasted_iota(jnp.int32, (d, f), 0)
            oh_ref[...] = jnp.where(di == idx_ref[0, :][None, :], 1.0, 0.0)

        o_ref[...] = jnp.dot(x_ref[...], oh_ref[...],
                             preferred_element_type=jnp.float32)

    return pl.pallas_call(
        body,
        grid=(nrows_tc // br,),
        in_specs=[
            pl.BlockSpec((8, f), lambda i: (0, 0)),
            pl.BlockSpec((br, d), lambda i: (i, 0)),
        ],
        out_specs=pl.BlockSpec((br, f), lambda i: (i, 0)),
        out_shape=jax.ShapeDtypeStruct((nrows_out, f), jnp.float32),
        scratch_shapes=[pltpu.VMEM((d, f), jnp.float32)],
    )


_SC_ROWS = 4096


def kernel(X, feature_indices):
    b, s, d = X.shape
    f = feature_indices.shape[0]
    nrows = b * s
    x2d = X.reshape(nrows, d)
    tc_rows = nrows - _SC_ROWS
    idx8 = jnp.broadcast_to(feature_indices, (8, f))
    out_sc = _build_sc_gather(_SC_ROWS, d, f, 16, tc_rows)(x2d, feature_indices)
    out_full = _build_tc_gather(tc_rows, d, f, 512, nrows)(idx8, x2d)
    out2d = lax.dynamic_update_slice(out_full, out_sc, (tc_rows, 0))
    return out2d.reshape(b, s, f)


# speedup vs baseline: 1.0846x; 1.0466x over previous
import functools

import jax
import jax.numpy as jnp
from jax import lax
from jax.experimental import pallas as pl
from jax.experimental.pallas import tpu as pltpu
from jax.experimental.pallas import tpu_sc as plsc

L = 16


@functools.lru_cache(maxsize=None)
def _build_sc_gather(sc_rows, d, f, chunk_rows, row_base):
    info = plsc.get_sparse_core_info()
    nc, ns = info.num_cores, info.num_subcores
    nw = nc * ns
    assert sc_rows % (nw * chunk_rows) == 0
    rows_per_worker = sc_rows // nw
    n_chunks = rows_per_worker // chunk_rows
    n_groups = f // L

    mesh = plsc.VectorSubcoreMesh(core_axis_name="c", subcore_axis_name="s")

    assert n_chunks % 2 == 0
    n_pairs = n_chunks // 2

    @functools.partial(
        pl.kernel,
        out_type=jax.ShapeDtypeStruct((sc_rows, f), jnp.float32),
        mesh=mesh,
        compiler_params=pltpu.CompilerParams(
            needs_layout_passes=False,
            skip_device_barrier=True,
            disable_bounds_checks=True,
            disable_semaphore_checks=True,
        ),
        scratch_types=[
            pltpu.VMEM((f,), jnp.int32),
            pltpu.VMEM((chunk_rows, d), jnp.float32),
            pltpu.VMEM((chunk_rows, d), jnp.float32),
            pltpu.VMEM((chunk_rows, f), jnp.float32),
            pltpu.VMEM((chunk_rows, f), jnp.float32),
            pltpu.SemaphoreType.DMA,
            pltpu.SemaphoreType.DMA,
            pltpu.SemaphoreType.DMA,
            pltpu.SemaphoreType.DMA,
        ],
    )
    def sc_gather(x_hbm, idx_hbm, out_hbm, idx_v, in0, in1, ob0, ob1,
                  si0, si1, so0, so1):
        wid = lax.axis_index("s") * nc + lax.axis_index("c")
        base = wid * rows_per_worker
        pltpu.sync_copy(idx_hbm, idx_v)
        col_idx = [idx_v[pl.ds(g * L, L)] for g in range(n_groups)]

        def cp_in(c, buf, sem):
            return pltpu.make_async_copy(
                x_hbm.at[pl.ds(row_base + base + c * chunk_rows, chunk_rows)],
                buf, sem)

        def cp_out(c, buf, sem):
            return pltpu.make_async_copy(
                buf, out_hbm.at[pl.ds(base + c * chunk_rows, chunk_rows)], sem)

        def subsample(in_b, out_b):
            def row_body(i, carry):
                row = jnp.full((L,), i, jnp.int32)
                for g in range(n_groups):
                    vals = plsc.load_gather(in_b, [row, col_idx[g]])
                    out_b[i, pl.ds(g * L, L)] = vals
                return carry

            lax.fori_loop(0, chunk_rows, row_body, 0)

        cp_in(0, in0, si0).start()
        cp_in(1, in1, si1).start()

        def body(k, carry):
            c0 = 2 * k
            cp_in(c0, in0, si0).wait()

            @pl.when(k > 0)
            def _():
                cp_out(c0 - 2, ob0, so0).wait()

            subsample(in0, ob0)
            cp_out(c0, ob0, so0).start()

            @pl.when(k < n_pairs - 1)
            def _():
                cp_in(c0 + 2, in0, si0).start()

            cp_in(c0 + 1, in1, si1).wait()

            @pl.when(k > 0)
            def _():
                cp_out(c0 - 1, ob1, so1).wait()

            subsample(in1, ob1)
            cp_out(c0 + 1, ob1, so1).start()

            @pl.when(k < n_pairs - 1)
            def _():
                cp_in(c0 + 3, in1, si1).start()

            return carry

        lax.fori_loop(0, n_pairs, body, 0)
        cp_out(n_chunks - 2, ob0, so0).wait()
        cp_out(n_chunks - 1, ob1, so1).wait()

    return sc_gather


@functools.lru_cache(maxsize=None)
def _build_tc_gather(nrows_tc, d, f, br, nrows_out):
    def body(idx_ref, x_ref, o_ref, oh_ref):
        @pl.when(pl.program_id(0) == 0)
        def _():
            di = lax.broadcasted_iota(jnp.int32, (d, f), 0)
            oh_ref[...] = jnp.where(di == idx_ref[0, :][None, :], 1.0, 0.0)

        o_ref[...] = jnp.dot(x_ref[...], oh_ref[...],
                             preferred_element_type=jnp.float32)

    return pl.pallas_call(
        body,
        grid=(nrows_tc // br,),
        in_specs=[
            pl.BlockSpec((8, f), lambda i: (0, 0)),
            pl.BlockSpec((br, d), lambda i: (i, 0)),
        ],
        out_specs=pl.BlockSpec((br, f), lambda i: (i, 0)),
        out_shape=jax.ShapeDtypeStruct((nrows_out, f), jnp.float32),
        scratch_shapes=[pltpu.VMEM((d, f), jnp.float32)],
    )


_SC_ROWS = 3072


def kernel(X, feature_indices):
    b, s, d = X.shape
    f = feature_indices.shape[0]
    nrows = b * s
    x2d = X.reshape(nrows, d)
    tc_rows = nrows - _SC_ROWS
    idx8 = jnp.broadcast_to(feature_indices, (8, f))
    out_sc = _build_sc_gather(_SC_ROWS, d, f, 16, tc_rows)(x2d, feature_indices)
    out_full = _build_tc_gather(tc_rows, d, f, 512, nrows)(idx8, x2d)
    out2d = lax.dynamic_update_slice(out_full, out_sc, (tc_rows, 0))
    return out2d.reshape(b, s, f)
